# Initial kernel scaffold; baseline (speedup 1.0000x reference)
import jax
import jax.numpy as jnp
from jax.experimental import pallas as pl


def kernel(x, A, w_aw, w_axw, W_att):
    raise NotImplementedError("write your pallas kernel here")


# speedup vs baseline: 1.3819x; 1.3819x over previous
import jax
import jax.numpy as jnp
from jax.experimental import pallas as pl
from jax.experimental.pallas import tpu as pltpu

N = 10000
D_IN = 128
DIM = 32
BLK = 400


def _pass1_kernel(A_ref, w_aw_ref, x_ref, w_axw_ref, W_att_ref, y_ref,
                  wcat_ref):
    @pl.when(pl.program_id(0) == 0)
    def _():
        wcat_ref[:, :DIM] = w_aw_ref[:]
        wcat_ref[:, DIM:] = jnp.dot(
            x_ref[:], w_axw_ref[:], preferred_element_type=jnp.float32)

    h = jnp.dot(A_ref[:], wcat_ref[:], preferred_element_type=jnp.float32)
    a = jnp.maximum(h[:, :DIM], 0.0)
    b = jnp.maximum(h[:, DIM:], 0.0)
    c = (a + b) * 0.5
    context = jnp.tanh(
        jnp.dot(c, W_att_ref[:], preferred_element_type=jnp.float32))
    s1 = jax.nn.sigmoid(jnp.sum(a * context, axis=1, keepdims=True)) + 1e-10
    s2 = jax.nn.sigmoid(jnp.sum(b * context, axis=1, keepdims=True)) + 1e-10
    inv = jax.lax.rsqrt(s1 * s1 + s2 * s2)
    y_ref[:] = a * (s1 * inv) + b * (s2 * inv)


def _pass2_kernel(A_ref, y_ref, out_ref):
    out_ref[:] = jnp.dot(A_ref[:], y_ref[:], preferred_element_type=jnp.float32)


def kernel(x, A, w_aw, w_axw, W_att):
    n_blocks = N // BLK

    y = pl.pallas_call(
        _pass1_kernel,
        grid=(n_blocks,),
        in_specs=[
            pl.BlockSpec((BLK, N), lambda i: (i, 0)),
            pl.BlockSpec((N, DIM), lambda i: (0, 0)),
            pl.BlockSpec((N, D_IN), lambda i: (0, 0)),
            pl.BlockSpec((D_IN, DIM), lambda i: (0, 0)),
            pl.BlockSpec((DIM, DIM), lambda i: (0, 0)),
        ],
        out_specs=pl.BlockSpec((BLK, DIM), lambda i: (i, 0)),
        out_shape=jax.ShapeDtypeStruct((N, DIM), jnp.float32),
        scratch_shapes=[pltpu.VMEM((N, 2 * DIM), jnp.float32)],
        compiler_params=pltpu.CompilerParams(
            dimension_semantics=("arbitrary",)),
    )(A, w_aw, x, w_axw, W_att)

    out = pl.pallas_call(
        _pass2_kernel,
        grid=(n_blocks,),
        in_specs=[
            pl.BlockSpec((BLK, N), lambda i: (i, 0)),
            pl.BlockSpec((N, DIM), lambda i: (0, 0)),
        ],
        out_specs=pl.BlockSpec((BLK, DIM), lambda i: (i, 0)),
        out_shape=jax.ShapeDtypeStruct((N, DIM), jnp.float32),
        compiler_params=pltpu.CompilerParams(
            dimension_semantics=("parallel",)),
    )(A, y)

    return out


# speedup vs baseline: 1.4392x; 1.0414x over previous
import jax
import jax.numpy as jnp
from jax.experimental import pallas as pl
from jax.experimental.pallas import tpu as pltpu

N = 10000
D_IN = 128
DIM = 32
BLK = 400


def _fused_kernel(A_ref, w_aw_ref, x_ref, w_axw_ref, W_att_ref, out_ref,
                  wcat_ref, y_ref):
    phase = pl.program_id(0)
    i = pl.program_id(1)

    @pl.when(jnp.logical_and(phase == 0, i == 0))
    def _():
        wcat_ref[:, :DIM] = w_aw_ref[:]
        wcat_ref[:, DIM:] = jnp.dot(
            x_ref[:], w_axw_ref[:], preferred_element_type=jnp.float32)

    @pl.when(phase == 0)
    def _():
        h = jnp.dot(A_ref[:], wcat_ref[:], preferred_element_type=jnp.float32)
        a = jnp.maximum(h[:, :DIM], 0.0)
        b = jnp.maximum(h[:, DIM:], 0.0)
        c = (a + b) * 0.5
        context = jnp.tanh(
            jnp.dot(c, W_att_ref[:], preferred_element_type=jnp.float32))
        s1 = jax.nn.sigmoid(
            jnp.sum(a * context, axis=1, keepdims=True)) + 1e-10
        s2 = jax.nn.sigmoid(
            jnp.sum(b * context, axis=1, keepdims=True)) + 1e-10
        inv = jax.lax.rsqrt(s1 * s1 + s2 * s2)
        y_ref[pl.ds(i * BLK, BLK), :] = a * (s1 * inv) + b * (s2 * inv)
        out_ref[:] = jnp.zeros_like(out_ref)

    @pl.when(phase == 1)
    def _():
        out_ref[:] = jnp.dot(
            A_ref[:], y_ref[:], preferred_element_type=jnp.float32)


def kernel(x, A, w_aw, w_axw, W_att):
    n_blocks = N // BLK

    out = pl.pallas_call(
        _fused_kernel,
        grid=(2, n_blocks),
        in_specs=[
            pl.BlockSpec((BLK, N), lambda p, i: (i, 0)),
            pl.BlockSpec((N, DIM), lambda p, i: (0, 0)),
            pl.BlockSpec((N, D_IN), lambda p, i: (0, 0)),
            pl.BlockSpec((D_IN, DIM), lambda p, i: (0, 0)),
            pl.BlockSpec((DIM, DIM), lambda p, i: (0, 0)),
        ],
        out_specs=pl.BlockSpec((BLK, DIM), lambda p, i: (i, 0)),
        out_shape=jax.ShapeDtypeStruct((N, DIM), jnp.float32),
        scratch_shapes=[
            pltpu.VMEM((N, 2 * DIM), jnp.float32),
            pltpu.VMEM((N, DIM), jnp.float32),
        ],
        compiler_params=pltpu.CompilerParams(
            dimension_semantics=("arbitrary", "arbitrary")),
    )(A, w_aw, x, w_axw, W_att)

    return out
